# baseline (speedup 1.0000x reference)
import functools

import jax
import jax.numpy as jnp
from jax import lax
from jax.experimental import pallas as pl
from jax.experimental.pallas import tpu as pltpu
from jax.experimental.pallas import tpu_sc as plsc

N = 10000
E = 320000
IN_DIM = 128
HID = 256
OUT = 256
NC_CLS = 10
L = 4

D = HID // 2
NSUB = 16
B = 128
NCHUNK = E // B
ITERS = -(-NCHUNK // NSUB)
NP = 10112
STRIPE = NP // NSUB
DW = 128


HALF_CH = NCHUNK // 2
DITERS = 80
CPAD = 2 * NSUB * DITERS


def _sc_degs_body(dstp_hbm, ones_hbm, zeros_hbm, out2_hbm, dst_v, ones_v,
                  ssem, acc):
    c = lax.axis_index("c")
    s = lax.axis_index("s")
    row0 = pl.multiple_of((c * NSUB + s) * DITERS, 8)
    pltpu.async_copy(dstp_hbm.at[pl.ds(row0, DITERS)], dst_v, ssem)
    pltpu.sync_copy(zeros_hbm, acc.at[pl.ds(s * STRIPE, STRIPE)])
    pltpu.sync_copy(ones_hbm, ones_v)
    pltpu.make_async_copy(
        dstp_hbm.at[pl.ds(row0, DITERS)], dst_v, ssem).wait()
    plsc.subcore_barrier()

    nvalid = jnp.minimum(jnp.maximum(HALF_CH - s * DITERS, 0), DITERS)

    def fire(j, carry):
        @pl.when(j < nvalid)
        def _():
            pltpu.async_copy(ones_v, acc.at[dst_v.at[j]], ssem, add=True)
        return carry

    lax.fori_loop(0, DITERS, fire, None)

    def drain(j, carry):
        @pl.when(j < nvalid)
        def _():
            pltpu.make_async_copy(
                ones_v, acc.at[dst_v.at[0]], ssem).wait()
        return carry

    lax.fori_loop(0, DITERS, drain, None)
    plsc.subcore_barrier()
    out_base = pl.multiple_of(c * NP + s * STRIPE, 8)
    pltpu.sync_copy(acc.at[pl.ds(s * STRIPE, STRIPE)],
                    out2_hbm.at[pl.ds(out_base, STRIPE)])


def _sc_agg_body(hn2_hbm, idx4_hbm, zeros_hbm, out2_hbm,
                 idx_v0, idx_v1, idx_v2, rows_v0, rows_v1, rows_v2,
                 gsem0, gsem1, gsem2, acc):
    c = lax.axis_index("c")
    s = lax.axis_index("s")
    pltpu.sync_copy(zeros_hbm, acc.at[pl.ds(s * STRIPE, STRIPE)])
    plsc.subcore_barrier()

    def cid_of(k):
        return k * NSUB + s

    def launch(k, idx_v, rows_v, gsem):
        @pl.when(cid_of(k) < NCHUNK)
        def _():
            pltpu.sync_copy(idx4_hbm.at[cid_of(k), pl.ds(2 * c, 2)], idx_v)
            pltpu.async_copy(hn2_hbm.at[idx_v.at[0]], rows_v, gsem)

    launch(0, idx_v0, rows_v0, gsem0)
    launch(1, idx_v1, rows_v1, gsem1)

    def stage(k, idx_a, rows_a, gsem_a, idx_c, rows_c, gsem_c):
        launch(k + 2, idx_c, rows_c, gsem_c)

        @pl.when(cid_of(k) < NCHUNK)
        def _():
            pltpu.make_async_copy(
                hn2_hbm.at[idx_a.at[0]], rows_a, gsem_a).wait()
            pltpu.sync_copy(rows_a, acc.at[idx_a.at[1]], add=True)

    def triple(j, carry):
        stage(3 * j, idx_v0, rows_v0, gsem0, idx_v2, rows_v2, gsem2)
        stage(3 * j + 1, idx_v1, rows_v1, gsem1, idx_v0, rows_v0, gsem0)
        stage(3 * j + 2, idx_v2, rows_v2, gsem2, idx_v1, rows_v1, gsem1)
        return carry

    lax.fori_loop(0, (ITERS + 2) // 3, triple, None)
    plsc.subcore_barrier()
    out_base = pl.multiple_of(c * NP + s * STRIPE, 8)
    pltpu.sync_copy(acc.at[pl.ds(s * STRIPE, STRIPE)],
                    out2_hbm.at[pl.ds(out_base, STRIPE)])


@functools.lru_cache(maxsize=1)
def _build_sc_kernels():
    mesh = plsc.VectorSubcoreMesh(
        core_axis_name="c", subcore_axis_name="s",
        num_cores=2, num_subcores=NSUB)
    degs_k = pl.kernel(
        _sc_degs_body,
        out_type=jax.ShapeDtypeStruct((2 * NP, DW), jnp.float32),
        mesh=mesh,
        scratch_types=[
            pltpu.VMEM((DITERS, B), jnp.int32),
            pltpu.VMEM((B, DW), jnp.float32),
            pltpu.SemaphoreType.DMA,
            pltpu.VMEM_SHARED((NP, DW), jnp.float32),
        ],
    )
    agg_k = pl.kernel(
        _sc_agg_body,
        out_type=jax.ShapeDtypeStruct((2 * NP, D), jnp.float32),
        mesh=mesh,
        scratch_types=[
            pltpu.VMEM((2, B), jnp.int32),
            pltpu.VMEM((2, B), jnp.int32),
            pltpu.VMEM((2, B), jnp.int32),
            pltpu.VMEM((B, D), jnp.float32),
            pltpu.VMEM((B, D), jnp.float32),
            pltpu.VMEM((B, D), jnp.float32),
            pltpu.SemaphoreType.DMA,
            pltpu.SemaphoreType.DMA,
            pltpu.SemaphoreType.DMA,
            pltpu.VMEM_SHARED((NP, D), jnp.float32),
        ],
    )
    return degs_k, agg_k


R = 1000
G = N // R


def _norm_col(degs_blk):
    d = degs_blk[0, :, 0:1] + degs_blk[1, :, 0:1]
    return lax.rsqrt(jnp.maximum(d, 1.0))


def _enc_body(h_ref, w_ref, b_ref, wp_ref, degs_ref,
              hcur_ref, hn2_ref, score_ref):
    hc = jnp.dot(h_ref[...], w_ref[...], preferred_element_type=jnp.float32)
    hc = hc + b_ref[...]
    hcur_ref[...] = hc
    hn = hc * _norm_col(degs_ref[...])
    hn2_ref[0] = hn[:, :D]
    hn2_ref[1] = hn[:, D:]
    score_ref[...] = jnp.dot(hc, wp_ref[...], preferred_element_type=jnp.float32)


_enc = pl.pallas_call(
    _enc_body,
    grid=(G,),
    in_specs=[
        pl.BlockSpec((R, IN_DIM), lambda i: (i, 0)),
        pl.BlockSpec((IN_DIM, HID), lambda i: (0, 0)),
        pl.BlockSpec((1, HID), lambda i: (0, 0)),
        pl.BlockSpec((OUT, 128), lambda i: (0, 0)),
        pl.BlockSpec((2, R, DW), lambda i: (0, i, 0)),
    ],
    out_specs=[
        pl.BlockSpec((R, HID), lambda i: (i, 0)),
        pl.BlockSpec((2, R, D), lambda i: (0, i, 0)),
        pl.BlockSpec((R, 128), lambda i: (i, 0)),
    ],
    out_shape=[
        jax.ShapeDtypeStruct((N, HID), jnp.float32),
        jax.ShapeDtypeStruct((2, N, D), jnp.float32),
        jax.ShapeDtypeStruct((N, 128), jnp.float32),
    ],
)


def _stats_body(hcur_ref, agg_ref, degs_ref, epsr_ref,
                rstp_ref, mom_ref):
    i = pl.program_id(0)
    hc = hcur_ref[...]
    agg = jnp.concatenate([agg_ref[0], agg_ref[1]], axis=1)
    r = hc * epsr_ref[...] + agg * _norm_col(degs_ref[...])
    rstp_ref[...] = r
    m = jnp.stack([jnp.sum(r, axis=0), jnp.sum(r * r, axis=0)])

    @pl.when(i == 0)
    def _():
        mom_ref[...] = m

    @pl.when(i > 0)
    def _():
        mom_ref[...] += m


_stats = pl.pallas_call(
    _stats_body,
    grid=(G,),
    in_specs=[
        pl.BlockSpec((R, HID), lambda i: (i, 0)),
        pl.BlockSpec((2, R, D), lambda i: (0, i, 0)),
        pl.BlockSpec((2, R, DW), lambda i: (0, i, 0)),
        pl.BlockSpec((1, HID), lambda i: (0, 0)),
    ],
    out_specs=[
        pl.BlockSpec((R, HID), lambda i: (i, 0)),
        pl.BlockSpec((2, HID), lambda i: (0, 0)),
    ],
    out_shape=[
        jax.ShapeDtypeStruct((N, HID), jnp.float32),
        jax.ShapeDtypeStruct((2, HID), jnp.float32),
    ],
)


def _layer_body(rstp_ref, hcur_ref, sc_ref, sh_ref, wl_ref, bl_ref,
                degs_ref,
                hn_ref, hn2_ref):
    r = jnp.maximum(rstp_ref[...] * sc_ref[...] + sh_ref[...], 0.0)
    r = r + hcur_ref[...]
    hn = jnp.dot(r, wl_ref[...], preferred_element_type=jnp.float32)
    hn = hn + bl_ref[...]
    hn_ref[...] = hn
    hnn = hn * _norm_col(degs_ref[...])
    hn2_ref[0] = hnn[:, :D]
    hn2_ref[1] = hnn[:, D:]


_layer = pl.pallas_call(
    _layer_body,
    grid=(G,),
    in_specs=[
        pl.BlockSpec((R, HID), lambda i: (i, 0)),
        pl.BlockSpec((R, HID), lambda i: (i, 0)),
        pl.BlockSpec((1, HID), lambda i: (0, 0)),
        pl.BlockSpec((1, HID), lambda i: (0, 0)),
        pl.BlockSpec((HID, OUT), lambda i: (0, 0)),
        pl.BlockSpec((1, OUT), lambda i: (0, 0)),
        pl.BlockSpec((2, R, DW), lambda i: (0, i, 0)),
    ],
    out_specs=[
        pl.BlockSpec((R, OUT), lambda i: (i, 0)),
        pl.BlockSpec((2, R, D), lambda i: (0, i, 0)),
    ],
    out_shape=[
        jax.ShapeDtypeStruct((N, OUT), jnp.float32),
        jax.ShapeDtypeStruct((2, N, D), jnp.float32),
    ],
)


def _score_body(hn_ref, wp_ref, score_ref, sco_ref):
    sco_ref[...] = score_ref[...] + jnp.dot(
        hn_ref[...], wp_ref[...], preferred_element_type=jnp.float32)


_score = pl.pallas_call(
    _score_body,
    grid=(G,),
    in_specs=[
        pl.BlockSpec((R, OUT), lambda i: (i, 0)),
        pl.BlockSpec((OUT, 128), lambda i: (0, 0)),
        pl.BlockSpec((R, 128), lambda i: (i, 0)),
    ],
    out_specs=pl.BlockSpec((R, 128), lambda i: (i, 0)),
    out_shape=jax.ShapeDtypeStruct((N, 128), jnp.float32),
    input_output_aliases={2: 0},
)


@jax.jit
def kernel(h, edge_index, e, W_enc, b_enc, eps, gamma, beta,
           W_lin, b_lin, W_pred, b_pred):
    src = edge_index[0]
    dst = edge_index[1]
    idx4 = jnp.stack([src, dst, src + N, dst]).reshape(
        4, NCHUNK, B).transpose(1, 0, 2)

    ones_dw = jnp.ones((B, DW), jnp.float32)
    zeros_dw = jnp.zeros((STRIPE, DW), jnp.float32)
    zeros_d = jnp.zeros((STRIPE, D), jnp.float32)
    wp_pad = jnp.pad(W_pred, ((0, 0), (0, 0), (0, 128 - NC_CLS)))
    b_total = jnp.sum(b_pred, axis=0)

    dstc = dst.reshape(NCHUNK, B)
    pad = jnp.zeros((NSUB * DITERS - HALF_CH, B), jnp.int32)
    dstp = jnp.concatenate([dstc[:HALF_CH], pad, dstc[HALF_CH:], pad])

    sc_degs, sc_agg = _build_sc_kernels()
    degs = sc_degs(dstp, ones_dw, zeros_dw).reshape(2, NP, DW)

    hcur, hn2, score = _enc(h, W_enc, b_enc.reshape(1, HID), wp_pad[0], degs)

    for i in range(L):
        agg2 = sc_agg(hn2.reshape(2 * N, D), idx4, zeros_d)
        agg2 = agg2.reshape(2, NP, D)
        epsr = jnp.full((1, HID), 1.0, jnp.float32) + eps[i]
        rstp, mom = _stats(hcur, agg2, degs, epsr)
        mean = mom[0] / N
        var = mom[1] / N - mean * mean
        ginv = gamma[i] * lax.rsqrt(var + 1e-5)
        scale = ginv.reshape(1, HID)
        shift = (beta[i] - mean * ginv).reshape(1, HID)
        hcur, hn2 = _layer(
            rstp, hcur, scale, shift, W_lin[i], b_lin[i].reshape(1, OUT),
            degs)
        score = _score(hcur, wp_pad[i + 1], score)

    return score[:, :NC_CLS] + b_total


# speedup vs baseline: 1.0054x; 1.0054x over previous
import functools

import jax
import jax.numpy as jnp
from jax import lax
from jax.experimental import pallas as pl
from jax.experimental.pallas import tpu as pltpu
from jax.experimental.pallas import tpu_sc as plsc

N = 10000
E = 320000
IN_DIM = 128
HID = 256
OUT = 256
NC_CLS = 10
L = 4

D = HID // 2
NSUB = 16
B = 128
NCHUNK = E // B
ITERS = -(-NCHUNK // NSUB)
NP = 10112
STRIPE = NP // NSUB
DW = 128


HALF_CH = NCHUNK // 2
DITERS = 80
CPAD = 2 * NSUB * DITERS


def _sc_degs_body(dstp_hbm, ones_hbm, zeros_hbm, out2_hbm, dst_v, ones_v,
                  ssem, acc):
    c = lax.axis_index("c")
    s = lax.axis_index("s")
    row0 = pl.multiple_of((c * NSUB + s) * DITERS, 8)
    pltpu.async_copy(dstp_hbm.at[pl.ds(row0, DITERS)], dst_v, ssem)
    pltpu.sync_copy(zeros_hbm, acc.at[pl.ds(s * STRIPE, STRIPE)])
    pltpu.sync_copy(ones_hbm, ones_v)
    pltpu.make_async_copy(
        dstp_hbm.at[pl.ds(row0, DITERS)], dst_v, ssem).wait()
    plsc.subcore_barrier()

    nvalid = jnp.minimum(jnp.maximum(HALF_CH - s * DITERS, 0), DITERS)

    def fire(j, carry):
        @pl.when(j < nvalid)
        def _():
            pltpu.async_copy(ones_v, acc.at[dst_v.at[j]], ssem, add=True)
        return carry

    lax.fori_loop(0, DITERS, fire, None)

    def drain(j, carry):
        @pl.when(j < nvalid)
        def _():
            pltpu.make_async_copy(
                ones_v, acc.at[dst_v.at[0]], ssem).wait()
        return carry

    lax.fori_loop(0, DITERS, drain, None)
    plsc.subcore_barrier()
    out_base = pl.multiple_of(c * NP + s * STRIPE, 8)
    pltpu.sync_copy(acc.at[pl.ds(s * STRIPE, STRIPE)],
                    out2_hbm.at[pl.ds(out_base, STRIPE)])


def _sc_agg_body(hn2_hbm, idx4_hbm, zeros_hbm, out2_hbm,
                 idx_v0, idx_v1, idx_v2, rows_v0, rows_v1, rows_v2,
                 gsem0, gsem1, gsem2, acc):
    c = lax.axis_index("c")
    s = lax.axis_index("s")
    pltpu.sync_copy(zeros_hbm, acc.at[pl.ds(s * STRIPE, STRIPE)])
    plsc.subcore_barrier()

    def cid_of(k):
        return k * NSUB + s

    def launch(k, idx_v, rows_v, gsem):
        @pl.when(cid_of(k) < NCHUNK)
        def _():
            pltpu.sync_copy(idx4_hbm.at[cid_of(k), pl.ds(2 * c, 2)], idx_v)
            pltpu.async_copy(hn2_hbm.at[idx_v.at[0]], rows_v, gsem)

    launch(0, idx_v0, rows_v0, gsem0)
    launch(1, idx_v1, rows_v1, gsem1)

    def stage(k, idx_a, rows_a, gsem_a, idx_c, rows_c, gsem_c):
        launch(k + 2, idx_c, rows_c, gsem_c)

        @pl.when(cid_of(k) < NCHUNK)
        def _():
            pltpu.make_async_copy(
                hn2_hbm.at[idx_a.at[0]], rows_a, gsem_a).wait()
            pltpu.sync_copy(rows_a, acc.at[idx_a.at[1]], add=True)

    def triple(j, carry):
        stage(3 * j, idx_v0, rows_v0, gsem0, idx_v2, rows_v2, gsem2)
        stage(3 * j + 1, idx_v1, rows_v1, gsem1, idx_v0, rows_v0, gsem0)
        stage(3 * j + 2, idx_v2, rows_v2, gsem2, idx_v1, rows_v1, gsem1)
        return carry

    lax.fori_loop(0, (ITERS + 2) // 3, triple, None)
    plsc.subcore_barrier()
    out_base = pl.multiple_of(c * NP + s * STRIPE, 8)
    pltpu.sync_copy(acc.at[pl.ds(s * STRIPE, STRIPE)],
                    out2_hbm.at[pl.ds(out_base, STRIPE)])


@functools.lru_cache(maxsize=1)
def _build_sc_kernels():
    mesh = plsc.VectorSubcoreMesh(
        core_axis_name="c", subcore_axis_name="s",
        num_cores=2, num_subcores=NSUB)
    degs_k = pl.kernel(
        _sc_degs_body,
        out_type=jax.ShapeDtypeStruct((2 * NP, DW), jnp.float32),
        mesh=mesh,
        scratch_types=[
            pltpu.VMEM((DITERS, B), jnp.int32),
            pltpu.VMEM((B, DW), jnp.float32),
            pltpu.SemaphoreType.DMA,
            pltpu.VMEM_SHARED((NP, DW), jnp.float32),
        ],
    )
    agg_k = pl.kernel(
        _sc_agg_body,
        out_type=jax.ShapeDtypeStruct((2 * NP, D), jnp.float32),
        mesh=mesh,
        scratch_types=[
            pltpu.VMEM((2, B), jnp.int32),
            pltpu.VMEM((2, B), jnp.int32),
            pltpu.VMEM((2, B), jnp.int32),
            pltpu.VMEM((B, D), jnp.float32),
            pltpu.VMEM((B, D), jnp.float32),
            pltpu.VMEM((B, D), jnp.float32),
            pltpu.SemaphoreType.DMA,
            pltpu.SemaphoreType.DMA,
            pltpu.SemaphoreType.DMA,
            pltpu.VMEM_SHARED((NP, D), jnp.float32),
        ],
    )
    return degs_k, agg_k


R = 1000
G = N // R


def _norm_col(degs_blk):
    d = degs_blk[0, :, 0:1] + degs_blk[1, :, 0:1]
    return lax.rsqrt(jnp.maximum(d, 1.0))


def _enc_body(h_ref, w_ref, b_ref, wp_ref, degs_ref,
              hcur_ref, hn2_ref, score_ref):
    hc = jnp.dot(h_ref[...], w_ref[...], preferred_element_type=jnp.float32)
    hc = hc + b_ref[...]
    hcur_ref[...] = hc
    hn = hc * _norm_col(degs_ref[...])
    hn2_ref[0] = hn[:, :D]
    hn2_ref[1] = hn[:, D:]
    score_ref[...] = jnp.dot(hc, wp_ref[...], preferred_element_type=jnp.float32)


_enc = pl.pallas_call(
    _enc_body,
    grid=(G,),
    in_specs=[
        pl.BlockSpec((R, IN_DIM), lambda i: (i, 0)),
        pl.BlockSpec((IN_DIM, HID), lambda i: (0, 0)),
        pl.BlockSpec((1, HID), lambda i: (0, 0)),
        pl.BlockSpec((OUT, 128), lambda i: (0, 0)),
        pl.BlockSpec((2, R, DW), lambda i: (0, i, 0)),
    ],
    out_specs=[
        pl.BlockSpec((R, HID), lambda i: (i, 0)),
        pl.BlockSpec((2, R, D), lambda i: (0, i, 0)),
        pl.BlockSpec((R, 128), lambda i: (i, 0)),
    ],
    out_shape=[
        jax.ShapeDtypeStruct((N, HID), jnp.float32),
        jax.ShapeDtypeStruct((2, N, D), jnp.float32),
        jax.ShapeDtypeStruct((N, 128), jnp.float32),
    ],
)


def _stats_body(hcur_ref, agg_ref, degs_ref, epsr_ref,
                rstp_ref, mom_ref):
    i = pl.program_id(0)
    hc = hcur_ref[...]
    agg = jnp.concatenate([agg_ref[0], agg_ref[1]], axis=1)
    r = hc * epsr_ref[...] + agg * _norm_col(degs_ref[...])
    rstp_ref[...] = r
    m = jnp.stack([jnp.sum(r, axis=0), jnp.sum(r * r, axis=0)])

    @pl.when(i == 0)
    def _():
        mom_ref[...] = m

    @pl.when(i > 0)
    def _():
        mom_ref[...] += m


_stats = pl.pallas_call(
    _stats_body,
    grid=(G,),
    in_specs=[
        pl.BlockSpec((R, HID), lambda i: (i, 0)),
        pl.BlockSpec((2, R, D), lambda i: (0, i, 0)),
        pl.BlockSpec((2, R, DW), lambda i: (0, i, 0)),
        pl.BlockSpec((1, HID), lambda i: (0, 0)),
    ],
    out_specs=[
        pl.BlockSpec((R, HID), lambda i: (i, 0)),
        pl.BlockSpec((2, HID), lambda i: (0, 0)),
    ],
    out_shape=[
        jax.ShapeDtypeStruct((N, HID), jnp.float32),
        jax.ShapeDtypeStruct((2, HID), jnp.float32),
    ],
)


def _layer_body(rstp_ref, hcur_ref, sc_ref, sh_ref, wl_ref, bl_ref, wp_ref,
                degs_ref, score_ref,
                hn_ref, hn2_ref, sco_ref):
    r = jnp.maximum(rstp_ref[...] * sc_ref[...] + sh_ref[...], 0.0)
    r = r + hcur_ref[...]
    hn = jnp.dot(r, wl_ref[...], preferred_element_type=jnp.float32)
    hn = hn + bl_ref[...]
    hn_ref[...] = hn
    hnn = hn * _norm_col(degs_ref[...])
    hn2_ref[0] = hnn[:, :D]
    hn2_ref[1] = hnn[:, D:]
    sco_ref[...] = score_ref[...] + jnp.dot(
        hn, wp_ref[...], preferred_element_type=jnp.float32)


_layer = pl.pallas_call(
    _layer_body,
    grid=(G,),
    in_specs=[
        pl.BlockSpec((R, HID), lambda i: (i, 0)),
        pl.BlockSpec((R, HID), lambda i: (i, 0)),
        pl.BlockSpec((1, HID), lambda i: (0, 0)),
        pl.BlockSpec((1, HID), lambda i: (0, 0)),
        pl.BlockSpec((HID, OUT), lambda i: (0, 0)),
        pl.BlockSpec((1, OUT), lambda i: (0, 0)),
        pl.BlockSpec((OUT, 128), lambda i: (0, 0)),
        pl.BlockSpec((2, R, DW), lambda i: (0, i, 0)),
        pl.BlockSpec((R, 128), lambda i: (i, 0)),
    ],
    out_specs=[
        pl.BlockSpec((R, OUT), lambda i: (i, 0)),
        pl.BlockSpec((2, R, D), lambda i: (0, i, 0)),
        pl.BlockSpec((R, 128), lambda i: (i, 0)),
    ],
    out_shape=[
        jax.ShapeDtypeStruct((N, OUT), jnp.float32),
        jax.ShapeDtypeStruct((2, N, D), jnp.float32),
        jax.ShapeDtypeStruct((N, 128), jnp.float32),
    ],
    input_output_aliases={8: 2},
)


@jax.jit
def kernel(h, edge_index, e, W_enc, b_enc, eps, gamma, beta,
           W_lin, b_lin, W_pred, b_pred):
    src = edge_index[0]
    dst = edge_index[1]
    idx4 = jnp.stack([src, dst, src + N, dst]).reshape(
        4, NCHUNK, B).transpose(1, 0, 2)

    ones_dw = jnp.ones((B, DW), jnp.float32)
    zeros_dw = jnp.zeros((STRIPE, DW), jnp.float32)
    zeros_d = jnp.zeros((STRIPE, D), jnp.float32)
    wp_pad = jnp.pad(W_pred, ((0, 0), (0, 0), (0, 128 - NC_CLS)))
    b_total = jnp.sum(b_pred, axis=0)

    dstc = dst.reshape(NCHUNK, B)
    pad = jnp.zeros((NSUB * DITERS - HALF_CH, B), jnp.int32)
    dstp = jnp.concatenate([dstc[:HALF_CH], pad, dstc[HALF_CH:], pad])

    sc_degs, sc_agg = _build_sc_kernels()
    degs = sc_degs(dstp, ones_dw, zeros_dw).reshape(2, NP, DW)

    hcur, hn2, score = _enc(h, W_enc, b_enc.reshape(1, HID), wp_pad[0], degs)

    for i in range(L):
        agg2 = sc_agg(hn2.reshape(2 * N, D), idx4, zeros_d)
        agg2 = agg2.reshape(2, NP, D)
        epsr = jnp.full((1, HID), 1.0, jnp.float32) + eps[i]
        rstp, mom = _stats(hcur, agg2, degs, epsr)
        mean = mom[0] / N
        var = mom[1] / N - mean * mean
        ginv = gamma[i] * lax.rsqrt(var + 1e-5)
        scale = ginv.reshape(1, HID)
        shift = (beta[i] - mean * ginv).reshape(1, HID)
        hcur, hn2, score = _layer(
            rstp, hcur, scale, shift, W_lin[i], b_lin[i].reshape(1, OUT),
            wp_pad[i + 1], degs, score)

    return score[:, :NC_CLS] + b_total
